# Initial kernel scaffold; baseline (speedup 1.0000x reference)
import functools

import jax
import jax.numpy as jnp
from jax import lax
from jax.experimental import pallas as pl
from jax.experimental.pallas import tpu as pltpu
from jax.experimental.pallas import tpu_sc as plsc

N = 10000
D = 128
E = 320000

NC = 2
NS = 16
NW = NC * NS
EPB = 128
NB_PER_W = -(-E // (EPB * NW))
E_PAD = NB_PER_W * NW * EPB
PAD = E_PAD - E
ACC_ROWS = 10240
TRASH = N
RPS = ACC_ROWS // NS
ZROWS = 64
DEGW = 16

BR = 1000

_MESH = dict(core_axis_name="c", subcore_axis_name="s")


def _sc_degree(dp):
    @functools.partial(
        pl.kernel,
        out_type=jax.ShapeDtypeStruct((NC, ACC_ROWS, DEGW), jnp.float32),
        mesh=plsc.VectorSubcoreMesh(**_MESH),
        scratch_types=[
            pltpu.VMEM((EPB,), jnp.int32),
            pltpu.VMEM((EPB, DEGW), jnp.float32),
            pltpu.VMEM((ZROWS, DEGW), jnp.float32),
            pltpu.VMEM_SHARED((ACC_ROWS, DEGW), jnp.float32),
        ],
    )
    def k(dp_hbm, out_hbm, didx, ones, zbuf, acc):
        cid = lax.axis_index("c")
        sid = lax.axis_index("s")
        gwid = cid * NS + sid

        @pl.loop(0, EPB)
        def _(r):
            ones[r, pl.ds(0, 16)] = jnp.ones((16,), jnp.float32)

        @pl.loop(0, ZROWS)
        def _(r):
            zbuf[r, pl.ds(0, 16)] = jnp.zeros((16,), jnp.float32)

        @pl.loop(0, RPS // ZROWS)
        def _(i):
            pltpu.sync_copy(zbuf, acc.at[pl.ds(sid * RPS + i * ZROWS, ZROWS)])

        plsc.subcore_barrier()

        @pl.loop(0, NB_PER_W)
        def _(b):
            base = (gwid * NB_PER_W + b) * EPB
            pltpu.sync_copy(dp_hbm.at[pl.ds(base, EPB)], didx)
            pltpu.sync_copy(ones, acc.at[didx], add=True)

        plsc.subcore_barrier()

        @pl.loop(0, RPS // ZROWS)
        def _(i):
            off = sid * RPS + i * ZROWS
            pltpu.sync_copy(acc.at[pl.ds(off, ZROWS)],
                            out_hbm.at[cid, pl.ds(off, ZROWS)])

    return k(dp)


def _sc_propagate(table, sp, dp):
    @functools.partial(
        pl.kernel,
        out_type=jax.ShapeDtypeStruct((NC, ACC_ROWS, D), jnp.float32),
        mesh=plsc.VectorSubcoreMesh(**_MESH),
        scratch_types=[
            pltpu.VMEM((EPB,), jnp.int32),
            pltpu.VMEM((EPB,), jnp.int32),
            pltpu.VMEM((EPB, D), jnp.float32),
            pltpu.VMEM((ZROWS, D), jnp.float32),
            pltpu.VMEM_SHARED((ACC_ROWS, D), jnp.float32),
        ],
    )
    def k(tab_hbm, sp_hbm, dp_hbm, out_hbm, sidx, didx, rows, zbuf, acc):
        cid = lax.axis_index("c")
        sid = lax.axis_index("s")
        gwid = cid * NS + sid

        @pl.loop(0, ZROWS)
        def _(r):
            @pl.loop(0, D // 16)
            def _(c):
                zbuf[r, pl.ds(c * 16, 16)] = jnp.zeros((16,), jnp.float32)

        @pl.loop(0, RPS // ZROWS)
        def _(i):
            pltpu.sync_copy(zbuf, acc.at[pl.ds(sid * RPS + i * ZROWS, ZROWS)])

        plsc.subcore_barrier()

        @pl.loop(0, NB_PER_W)
        def _(b):
            base = (gwid * NB_PER_W + b) * EPB
            pltpu.sync_copy(sp_hbm.at[pl.ds(base, EPB)], sidx)
            pltpu.sync_copy(dp_hbm.at[pl.ds(base, EPB)], didx)
            pltpu.sync_copy(tab_hbm.at[sidx], rows)
            pltpu.sync_copy(rows, acc.at[didx], add=True)

        plsc.subcore_barrier()

        @pl.loop(0, RPS // ZROWS)
        def _(i):
            off = sid * RPS + i * ZROWS
            pltpu.sync_copy(acc.at[pl.ds(off, ZROWS)],
                            out_hbm.at[cid, pl.ds(off, ZROWS)])

    return k(table, sp, dp)


def _tc_pre(accd, x):
    def body(accd_ref, x_ref, xs_ref, d_ref):
        deg = accd_ref[0, :, 0:1] + accd_ref[1, :, 0:1] + 1.0
        d = lax.rsqrt(deg)
        d_ref[...] = d
        xs_ref[...] = x_ref[...] * d

    return pl.pallas_call(
        body,
        grid=(N // BR,),
        in_specs=[
            pl.BlockSpec((NC, BR, DEGW), lambda i: (0, i, 0)),
            pl.BlockSpec((BR, D), lambda i: (i, 0)),
        ],
        out_specs=[
            pl.BlockSpec((BR, D), lambda i: (i, 0)),
            pl.BlockSpec((BR, 1), lambda i: (i, 0)),
        ],
        out_shape=[
            jax.ShapeDtypeStruct((N, D), jnp.float32),
            jax.ShapeDtypeStruct((N, 1), jnp.float32),
        ],
    )(accd, x)


def _tc_mid(acc, xs, d, W1, b1):
    def body(acc_ref, xs_ref, d_ref, w_ref, b_ref, emb_ref, es_ref):
        d = d_ref[...]
        h = (acc_ref[0] + acc_ref[1] + xs_ref[...]) * d
        e = lax.dot_general(h, w_ref[...],
                            dimension_numbers=(((1,), (1,)), ((), ())),
                            preferred_element_type=jnp.float32)
        e = jnp.maximum(e + b_ref[...], 0.0)
        emb_ref[...] = e
        es_ref[...] = e * d

    return pl.pallas_call(
        body,
        grid=(N // BR,),
        in_specs=[
            pl.BlockSpec((NC, BR, D), lambda i: (0, i, 0)),
            pl.BlockSpec((BR, D), lambda i: (i, 0)),
            pl.BlockSpec((BR, 1), lambda i: (i, 0)),
            pl.BlockSpec((D, D), lambda i: (0, 0)),
            pl.BlockSpec((1, D), lambda i: (0, 0)),
        ],
        out_specs=[
            pl.BlockSpec((BR, D), lambda i: (i, 0)),
            pl.BlockSpec((BR, D), lambda i: (i, 0)),
        ],
        out_shape=[
            jax.ShapeDtypeStruct((N, D), jnp.float32),
            jax.ShapeDtypeStruct((N, D), jnp.float32),
        ],
    )(acc, xs, d, W1, b1)


def _tc_post(acc, es, d, W2, b2):
    def body(acc_ref, es_ref, d_ref, w_ref, b_ref, out_ref):
        h = (acc_ref[0] + acc_ref[1] + es_ref[...]) * d_ref[...]
        o = lax.dot_general(h, w_ref[...],
                            dimension_numbers=(((1,), (1,)), ((), ())),
                            preferred_element_type=jnp.float32)
        out_ref[...] = o + b_ref[...]

    return pl.pallas_call(
        body,
        grid=(N // BR,),
        in_specs=[
            pl.BlockSpec((NC, BR, D), lambda i: (0, i, 0)),
            pl.BlockSpec((BR, D), lambda i: (i, 0)),
            pl.BlockSpec((BR, 1), lambda i: (i, 0)),
            pl.BlockSpec((D, D), lambda i: (0, 0)),
            pl.BlockSpec((1, D), lambda i: (0, 0)),
        ],
        out_specs=pl.BlockSpec((BR, D), lambda i: (i, 0)),
        out_shape=jax.ShapeDtypeStruct((N, D), jnp.float32),
    )(acc, es, d, W2, b2)


def kernel(x, edge_index, W1, b1, W2, b2):
    sp = jnp.concatenate([edge_index[0], jnp.zeros((PAD,), jnp.int32)])
    dp = jnp.concatenate([edge_index[1], jnp.full((PAD,), TRASH, jnp.int32)])
    accd = _sc_degree(dp)
    xs, d = _tc_pre(accd, x)
    acc1 = _sc_propagate(xs, sp, dp)
    emb, es = _tc_mid(acc1, xs, d, W1, b1.reshape(1, D))
    acc2 = _sc_propagate(es, sp, dp)
    out = _tc_post(acc2, es, d, W2, b2.reshape(1, D))
    return emb, out


# speedup vs baseline: 10.5125x; 10.5125x over previous
import functools

import jax
import jax.numpy as jnp
from jax import lax
from jax.experimental import pallas as pl
from jax.experimental.pallas import tpu as pltpu
from jax.experimental.pallas import tpu_sc as plsc

N = 10000
D = 128
E = 320000

NC = 2
NS = 16
NW = NC * NS
EPB = 128
NB_PER_W = -(-E // (EPB * NW))
E_PAD = NB_PER_W * NW * EPB
PAD = E_PAD - E
ACC_ROWS = 10240
TRASH = N
RPS = ACC_ROWS // NS
ZROWS = 64
DEGW = 128

BR = 1000

_MESH = dict(core_axis_name="c", subcore_axis_name="s")


def _sc_degree(dp):
    @functools.partial(
        pl.kernel,
        out_type=jax.ShapeDtypeStruct((NC, ACC_ROWS, DEGW), jnp.float32),
        mesh=plsc.VectorSubcoreMesh(**_MESH),
        scratch_types=[
            pltpu.VMEM((EPB,), jnp.int32),
            pltpu.VMEM((EPB, DEGW), jnp.float32),
            pltpu.VMEM_SHARED((ACC_ROWS, DEGW), jnp.float32),
        ],
    )
    def k(dp_hbm, out_hbm, didx, ones, acc):
        cid = lax.axis_index("c")
        sid = lax.axis_index("s")
        gwid = cid * NS + sid

        @pl.loop(0, EPB)
        def _(r):
            @pl.loop(0, DEGW // 16)
            def _(c):
                ones[r, pl.ds(c * 16, 16)] = jnp.ones((16,), jnp.float32)

        @pl.loop(0, RPS // ZROWS)
        def _(i):
            pltpu.sync_copy(ones.at[pl.ds(0, ZROWS)],
                            acc.at[pl.ds(sid * RPS + i * ZROWS, ZROWS)])

        plsc.subcore_barrier()

        @pl.loop(0, NB_PER_W)
        def _(b):
            base = (gwid * NB_PER_W + b) * EPB
            pltpu.sync_copy(dp_hbm.at[pl.ds(base, EPB)], didx)
            pltpu.sync_copy(ones, acc.at[didx], add=True)

        plsc.subcore_barrier()

        @pl.loop(0, RPS // ZROWS)
        def _(i):
            off = sid * RPS + i * ZROWS
            pltpu.sync_copy(acc.at[pl.ds(off, ZROWS)],
                            out_hbm.at[cid, pl.ds(off, ZROWS)])

    return k(dp)


def _sc_propagate(table, sp, dp):
    @functools.partial(
        pl.kernel,
        out_type=jax.ShapeDtypeStruct((NC, ACC_ROWS, D), jnp.float32),
        mesh=plsc.VectorSubcoreMesh(**_MESH),
        scratch_types=[
            pltpu.VMEM((EPB,), jnp.int32),
            pltpu.VMEM((EPB,), jnp.int32),
            pltpu.VMEM((EPB, D), jnp.float32),
            pltpu.VMEM((ZROWS, D), jnp.float32),
            pltpu.VMEM_SHARED((ACC_ROWS, D), jnp.float32),
        ],
    )
    def k(tab_hbm, sp_hbm, dp_hbm, out_hbm, sidx, didx, rows, zbuf, acc):
        cid = lax.axis_index("c")
        sid = lax.axis_index("s")
        gwid = cid * NS + sid

        @pl.loop(0, ZROWS)
        def _(r):
            @pl.loop(0, D // 16)
            def _(c):
                zbuf[r, pl.ds(c * 16, 16)] = jnp.zeros((16,), jnp.float32)

        @pl.loop(0, RPS // ZROWS)
        def _(i):
            pltpu.sync_copy(zbuf, acc.at[pl.ds(sid * RPS + i * ZROWS, ZROWS)])

        plsc.subcore_barrier()

        @pl.loop(0, NB_PER_W)
        def _(b):
            base = (gwid * NB_PER_W + b) * EPB
            pltpu.sync_copy(sp_hbm.at[pl.ds(base, EPB)], sidx)
            pltpu.sync_copy(dp_hbm.at[pl.ds(base, EPB)], didx)
            pltpu.sync_copy(tab_hbm.at[sidx], rows)
            pltpu.sync_copy(rows, acc.at[didx], add=True)

        plsc.subcore_barrier()

        @pl.loop(0, RPS // ZROWS)
        def _(i):
            off = sid * RPS + i * ZROWS
            pltpu.sync_copy(acc.at[pl.ds(off, ZROWS)],
                            out_hbm.at[cid, pl.ds(off, ZROWS)])

    return k(table, sp, dp)


def _tc_pre(accd, x):
    def body(accd_ref, x_ref, xs_ref, d_ref):
        deg = accd_ref[0, :, 0:1] + accd_ref[1, :, 0:1] - 1.0
        d = lax.rsqrt(deg)
        d_ref[...] = d
        xs_ref[...] = x_ref[...] * d

    return pl.pallas_call(
        body,
        grid=(N // BR,),
        in_specs=[
            pl.BlockSpec((NC, BR, DEGW), lambda i: (0, i, 0)),
            pl.BlockSpec((BR, D), lambda i: (i, 0)),
        ],
        out_specs=[
            pl.BlockSpec((BR, D), lambda i: (i, 0)),
            pl.BlockSpec((BR, 1), lambda i: (i, 0)),
        ],
        out_shape=[
            jax.ShapeDtypeStruct((N, D), jnp.float32),
            jax.ShapeDtypeStruct((N, 1), jnp.float32),
        ],
    )(accd, x)


def _tc_mid(acc, xs, d, W1, b1):
    def body(acc_ref, xs_ref, d_ref, w_ref, b_ref, emb_ref, es_ref):
        d = d_ref[...]
        h = (acc_ref[0] + acc_ref[1] + xs_ref[...]) * d
        e = lax.dot_general(h, w_ref[...],
                            dimension_numbers=(((1,), (1,)), ((), ())),
                            preferred_element_type=jnp.float32)
        e = jnp.maximum(e + b_ref[...], 0.0)
        emb_ref[...] = e
        es_ref[...] = e * d

    return pl.pallas_call(
        body,
        grid=(N // BR,),
        in_specs=[
            pl.BlockSpec((NC, BR, D), lambda i: (0, i, 0)),
            pl.BlockSpec((BR, D), lambda i: (i, 0)),
            pl.BlockSpec((BR, 1), lambda i: (i, 0)),
            pl.BlockSpec((D, D), lambda i: (0, 0)),
            pl.BlockSpec((1, D), lambda i: (0, 0)),
        ],
        out_specs=[
            pl.BlockSpec((BR, D), lambda i: (i, 0)),
            pl.BlockSpec((BR, D), lambda i: (i, 0)),
        ],
        out_shape=[
            jax.ShapeDtypeStruct((N, D), jnp.float32),
            jax.ShapeDtypeStruct((N, D), jnp.float32),
        ],
    )(acc, xs, d, W1, b1)


def _tc_post(acc, es, d, W2, b2):
    def body(acc_ref, es_ref, d_ref, w_ref, b_ref, out_ref):
        h = (acc_ref[0] + acc_ref[1] + es_ref[...]) * d_ref[...]
        o = lax.dot_general(h, w_ref[...],
                            dimension_numbers=(((1,), (1,)), ((), ())),
                            preferred_element_type=jnp.float32)
        out_ref[...] = o + b_ref[...]

    return pl.pallas_call(
        body,
        grid=(N // BR,),
        in_specs=[
            pl.BlockSpec((NC, BR, D), lambda i: (0, i, 0)),
            pl.BlockSpec((BR, D), lambda i: (i, 0)),
            pl.BlockSpec((BR, 1), lambda i: (i, 0)),
            pl.BlockSpec((D, D), lambda i: (0, 0)),
            pl.BlockSpec((1, D), lambda i: (0, 0)),
        ],
        out_specs=pl.BlockSpec((BR, D), lambda i: (i, 0)),
        out_shape=jax.ShapeDtypeStruct((N, D), jnp.float32),
    )(acc, es, d, W2, b2)


def kernel(x, edge_index, W1, b1, W2, b2):
    sp = jnp.concatenate([edge_index[0], jnp.zeros((PAD,), jnp.int32)])
    dp = jnp.concatenate([edge_index[1], jnp.full((PAD,), TRASH, jnp.int32)])
    accd = _sc_degree(dp)
    xs, d = _tc_pre(accd, x)
    acc1 = _sc_propagate(xs, sp, dp)
    emb, es = _tc_mid(acc1, xs, d, W1, b1.reshape(1, D))
    acc2 = _sc_propagate(es, sp, dp)
    out = _tc_post(acc2, es, d, W2, b2.reshape(1, D))
    return emb, out
